# Initial kernel scaffold; baseline (speedup 1.0000x reference)
import jax
import jax.numpy as jnp
from jax.experimental import pallas as pl


def kernel(x, sampled_tf):
    raise NotImplementedError("write your pallas kernel here")


# speedup vs baseline: 3.1776x; 3.1776x over previous
import dataclasses
import functools

import jax
import jax.numpy as jnp
from jax import lax
from jax.experimental import pallas as pl
from jax.experimental.pallas import tpu as pltpu
from jax.experimental.pallas import tpu_sc as plsc

NB_TF = 4
B, C, H, W = 256, 3, 224, 224
ROWS = B * C * H
L = 16
NC, NS = 2, 16
NW = NC * NS
IMGS_PER_TILE = B // NW
W_CHUNKS = W // L
HALF = H // 2


def _body(x_hbm, tf_hbm, o_hbm, chan_v, mn_v, mx_v, tf_v, sem):
    wid = lax.axis_index("c") * NS + lax.axis_index("s")

    pltpu.sync_copy(tf_hbm, tf_v)
    tf_chunk = tf_v[pl.ds((wid // 2) * L, L)]
    lane = lax.iota(jnp.int32, L)

    @pl.loop(0, IMGS_PER_TILE)
    def _(li):
        img = wid * IMGS_PER_TILE + li
        pos = (wid % 2) * IMGS_PER_TILE + li
        t = jnp.sum(jnp.where(lane == pos, tf_chunk, 0))

        @pl.loop(0, C)
        def _(ch):
            rb = (img * C + ch) * H

            pltpu.sync_copy(x_hbm.at[pl.ds(rb, H)], chan_v)

            @pl.when(t == 2)
            def _():
                @pl.loop(0, HALF)
                def _(h):
                    for j in range(W_CHUNKS):
                        a = chan_v[h, pl.ds(j * L, L)]
                        b = chan_v[H - 1 - h, pl.ds(j * L, L)]
                        chan_v[h, pl.ds(j * L, L)] = b
                        chan_v[H - 1 - h, pl.ds(j * L, L)] = a

            @pl.when(t == 1)
            def _():
                @pl.loop(0, H)
                def _(h):
                    for j in range(W_CHUNKS // 2):
                        a = chan_v[h, pl.ds(j * L, L)]
                        b = chan_v[h, pl.ds((W_CHUNKS - 1 - j) * L, L)]
                        chan_v[h, pl.ds(j * L, L)] = lax.rev(b, (0,))
                        chan_v[h, pl.ds((W_CHUNKS - 1 - j) * L, L)] = lax.rev(a, (0,))

            @pl.when(t == 3)
            def _():
                mn_v[...] = jnp.full((L,), jnp.inf, jnp.float32)
                mx_v[...] = jnp.full((L,), -jnp.inf, jnp.float32)

                @pl.loop(0, H)
                def _(h):
                    row_mn = chan_v[h, pl.ds(0, L)]
                    row_mx = row_mn
                    for j in range(1, W_CHUNKS):
                        v = chan_v[h, pl.ds(j * L, L)]
                        row_mn = jnp.minimum(row_mn, v)
                        row_mx = jnp.maximum(row_mx, v)
                    mn_v[...] = jnp.minimum(mn_v[...], row_mn)
                    mx_v[...] = jnp.maximum(mx_v[...], row_mx)

                mn = jnp.min(mn_v[...])
                mx = jnp.max(mx_v[...])
                mnb = jnp.full((L,), mn, jnp.float32)
                mxb = jnp.full((L,), mx, jnp.float32)
                scb = jnp.full((L,), 1.0, jnp.float32) / jnp.maximum(
                    mxb - mnb, jnp.full((L,), 1e-6, jnp.float32)
                )

                @pl.loop(0, H)
                def _(h):
                    for j in range(W_CHUNKS):
                        chan_v[h, pl.ds(j * L, L)] = (
                            chan_v[h, pl.ds(j * L, L)] - mnb
                        ) * scb

            pltpu.sync_copy(chan_v, o_hbm.at[pl.ds(rb, H)])


@jax.jit
def kernel(x, sampled_tf):
    x2 = x.reshape(ROWS, W)
    mesh = plsc.VectorSubcoreMesh(
        core_axis_name="c", subcore_axis_name="s", num_cores=NC, num_subcores=NS
    )
    cp = pltpu.CompilerParams()
    if "needs_layout_passes" in pltpu.CompilerParams.__dataclass_fields__:
        cp = dataclasses.replace(cp, needs_layout_passes=False)
    out2 = pl.kernel(
        _body,
        out_type=jax.ShapeDtypeStruct((ROWS, W), jnp.float32),
        mesh=mesh,
        scratch_types=[
            pltpu.VMEM((H, W), jnp.float32),
            pltpu.VMEM((L,), jnp.float32),
            pltpu.VMEM((L,), jnp.float32),
            pltpu.VMEM((B,), jnp.int32),
            pltpu.SemaphoreType.DMA,
        ],
        compiler_params=cp,
    )(x2, sampled_tf)
    return out2.reshape(B, C, H, W)


# speedup vs baseline: 3.4762x; 1.0940x over previous
import dataclasses

import jax
import jax.numpy as jnp
from jax import lax
from jax.experimental import pallas as pl
from jax.experimental.pallas import tpu as pltpu
from jax.experimental.pallas import tpu_sc as plsc

NB_TF = 4
B, C, H, W = 256, 3, 224, 224
ROWS = B * C * H
L = 16
NC, NS = 2, 16
NW = NC * NS
IMGS_PER_TILE = B // NW
SLOTS = IMGS_PER_TILE * C
W_CHUNKS = W // L
HALF = H // 2


def _transform(buf, t, mn_v, mx_v):
    @pl.when(t == 2)
    def _():
        @pl.loop(0, HALF)
        def _(h):
            for j in range(W_CHUNKS):
                a = buf[h, pl.ds(j * L, L)]
                b = buf[H - 1 - h, pl.ds(j * L, L)]
                buf[h, pl.ds(j * L, L)] = b
                buf[H - 1 - h, pl.ds(j * L, L)] = a

    @pl.when(t == 1)
    def _():
        @pl.loop(0, H)
        def _(h):
            for j in range(W_CHUNKS // 2):
                a = buf[h, pl.ds(j * L, L)]
                b = buf[h, pl.ds((W_CHUNKS - 1 - j) * L, L)]
                buf[h, pl.ds(j * L, L)] = lax.rev(b, (0,))
                buf[h, pl.ds((W_CHUNKS - 1 - j) * L, L)] = lax.rev(a, (0,))

    @pl.when(t == 3)
    def _():
        mn_v[...] = jnp.full((L,), jnp.inf, jnp.float32)
        mx_v[...] = jnp.full((L,), -jnp.inf, jnp.float32)

        @pl.loop(0, H)
        def _(h):
            row_mn = buf[h, pl.ds(0, L)]
            row_mx = row_mn
            for j in range(1, W_CHUNKS):
                v = buf[h, pl.ds(j * L, L)]
                row_mn = jnp.minimum(row_mn, v)
                row_mx = jnp.maximum(row_mx, v)
            mn_v[...] = jnp.minimum(mn_v[...], row_mn)
            mx_v[...] = jnp.maximum(mx_v[...], row_mx)

        mn = jnp.min(mn_v[...])
        mx = jnp.max(mx_v[...])
        mnb = jnp.full((L,), mn, jnp.float32)
        mxb = jnp.full((L,), mx, jnp.float32)
        scb = jnp.full((L,), 1.0, jnp.float32) / jnp.maximum(
            mxb - mnb, jnp.full((L,), 1e-6, jnp.float32)
        )

        @pl.loop(0, H)
        def _(h):
            for j in range(W_CHUNKS):
                buf[h, pl.ds(j * L, L)] = (buf[h, pl.ds(j * L, L)] - mnb) * scb


def _body(x_hbm, tf_hbm, o_hbm, buf0, buf1, mn_v, mx_v, tf_v,
          isem0, isem1, osem0, osem1):
    wid = lax.axis_index("c") * NS + lax.axis_index("s")
    base_slot = wid * SLOTS

    bufs = (buf0, buf1)
    isems = (isem0, isem1)
    osems = (osem0, osem1)

    def load(s, b):
        pltpu.async_copy(x_hbm.at[pl.ds((base_slot + s) * H, H)], bufs[b], isems[b])

    def store(s, b):
        pltpu.async_copy(bufs[b], o_hbm.at[pl.ds((base_slot + s) * H, H)], osems[b])

    def wait_load(b):
        pltpu.make_async_copy(x_hbm.at[pl.ds(0, H)], bufs[b], isems[b]).wait()

    def wait_store(b):
        pltpu.make_async_copy(bufs[b], o_hbm.at[pl.ds(0, H)], osems[b]).wait()

    pltpu.sync_copy(tf_hbm, tf_v)
    tf_chunk = tf_v[pl.ds((wid // 2) * L, L)]
    lane = lax.iota(jnp.int32, L)
    pos0 = (wid % 2) * IMGS_PER_TILE

    load(0, 0)

    @pl.loop(0, SLOTS // 6)
    def _(g):
        for j in range(6):
            s = 6 * g + j
            b = j % 2
            li = 2 * g + j // 3
            t = jnp.sum(jnp.where(lane == pos0 + li, tf_chunk, 0))

            wait_load(b)
            _transform(bufs[b], t, mn_v, mx_v)

            if j == 0:
                @pl.when(g > 0)
                def _():
                    wait_store(1 - b)
            else:
                wait_store(1 - b)
            if j == 5:
                @pl.when(g < SLOTS // 6 - 1)
                def _():
                    load(s + 1, 1 - b)
            else:
                load(s + 1, 1 - b)

            store(s, b)

    wait_store(1)


@jax.jit
def kernel(x, sampled_tf):
    x2 = x.reshape(ROWS, W)
    mesh = plsc.VectorSubcoreMesh(
        core_axis_name="c", subcore_axis_name="s", num_cores=NC, num_subcores=NS
    )
    cp = pltpu.CompilerParams()
    if "needs_layout_passes" in pltpu.CompilerParams.__dataclass_fields__:
        cp = dataclasses.replace(cp, needs_layout_passes=False)
    out2 = pl.kernel(
        _body,
        out_type=jax.ShapeDtypeStruct((ROWS, W), jnp.float32),
        mesh=mesh,
        scratch_types=[
            pltpu.VMEM((H, W), jnp.float32),
            pltpu.VMEM((H, W), jnp.float32),
            pltpu.VMEM((L,), jnp.float32),
            pltpu.VMEM((L,), jnp.float32),
            pltpu.VMEM((B,), jnp.int32),
            pltpu.SemaphoreType.DMA,
            pltpu.SemaphoreType.DMA,
            pltpu.SemaphoreType.DMA,
            pltpu.SemaphoreType.DMA,
        ],
        compiler_params=cp,
    )(x2, sampled_tf)
    return out2.reshape(B, C, H, W)
